# baseline (speedup 1.0000x reference)
import functools

import jax
import jax.numpy as jnp
from jax import lax
from jax.experimental import pallas as pl
from jax.experimental.pallas import tpu as pltpu
from jax.experimental.pallas import tpu_sc as plsc

B = 16
C = 3
H = 224
WID = 224
G = 14
P = 16
S = G * G
PATCH = C * P * P
ROWS = B * S
MAX_PIX = 400

NC = 2
NS = 16
NW = NC * NS
TASKS = B * G
TPW = TASKS // NW


def _gather_body(img_hbm, xp_hbm, band, sbuf, sem_in, sem_out):
    wid = lax.axis_index("s") * NC + lax.axis_index("c")

    @pl.when(wid < G * 2)
    def _():
        sy = wid % G
        bh = wid // G

        def load(bl, slot):
            return pltpu.async_copy(
                img_hbm.at[8 * bh + bl, :, pl.ds(sy * P, P), :],
                band.at[slot], sem_in)

        loads = [load(0, 0), load(1, 1), load(2, 2)] + [None] * 5
        for bl in range(8):
            loads[bl].wait()
            slot = bl % 3

            def retile(sx, carry, slot=slot, bl=bl):
                vals = [band[slot, c, y, pl.ds(sx * P, P)]
                        for c in range(C) for y in range(P)]
                for j, v in enumerate(vals):
                    sbuf[sx, bl, pl.ds(j * P, P)] = v
                return carry

            lax.fori_loop(0, G, retile, 0)
            if bl < 5:
                loads[bl + 3] = load(bl + 3, slot)

        pltpu.async_copy(
            sbuf, xp_hbm.at[pl.ds(sy * G, G), pl.ds(8 * bh, 8), :], sem_out
        ).wait()


_gather_sc = functools.partial(
    pl.kernel,
    mesh=plsc.VectorSubcoreMesh(core_axis_name="c", subcore_axis_name="s"),
    out_type=jax.ShapeDtypeStruct((S, B, PATCH), jnp.float32),
    scratch_types=[
        pltpu.VMEM((3, C, P, WID), jnp.float32),
        pltpu.VMEM((G, 8, PATCH), jnp.float32),
        pltpu.SemaphoreType.DMA,
        pltpu.SemaphoreType.DMA,
    ],
)(_gather_body)


def _matmul_body(x_ref, w_ref, b_ref, o_ref):
    acc = lax.dot_general(
        x_ref[...], w_ref[...], (((1,), (1,)), ((), ())),
        preferred_element_type=jnp.float32)
    o_ref[...] = acc + b_ref[...]


def _project_tc(x, wc, bias):
    m_blk = 1568
    grid = (ROWS // m_blk,)
    return pl.pallas_call(
        _matmul_body,
        grid=grid,
        in_specs=[
            pl.BlockSpec((m_blk, PATCH), lambda i: (i, 0)),
            pl.BlockSpec((PATCH, PATCH), lambda i: (0, 0)),
            pl.BlockSpec((1, PATCH), lambda i: (0, 0)),
        ],
        out_specs=pl.BlockSpec((m_blk, PATCH), lambda i: (i, 0)),
        out_shape=jax.ShapeDtypeStruct((ROWS, PATCH), jnp.float32),
    )(x, wc, bias)


def kernel(img, W, b):
    wc = W.reshape(-1, C, MAX_PIX)[:, :, : P * P].reshape(-1, PATCH)
    x = _gather_sc(img).reshape(ROWS, PATCH)
    out = _project_tc(x, wc, b.reshape(1, -1))
    return out.reshape(S, B, -1).transpose(1, 0, 2)


# speedup vs baseline: 1.0007x; 1.0007x over previous
import functools

import jax
import jax.numpy as jnp
from jax import lax
from jax.experimental import pallas as pl
from jax.experimental.pallas import tpu as pltpu
from jax.experimental.pallas import tpu_sc as plsc

B = 16
C = 3
H = 224
WID = 224
G = 14
P = 16
S = G * G
PATCH = C * P * P
ROWS = B * S
MAX_PIX = 400

NC = 2
NS = 16


def _gather_body(img_hbm, xp_hbm, band, sbuf, sem_in, sem_out):
    wid = lax.axis_index("s") * NC + lax.axis_index("c")

    @pl.when(wid < G * 2)
    def _():
        sy = wid % G
        bh = wid // G

        def load(bl, slot):
            return pltpu.async_copy(
                img_hbm.at[8 * bh + bl, :, pl.ds(sy * P, P), :],
                band.at[slot], sem_in)

        loads = [load(0, 0), load(1, 1), load(2, 2)] + [None] * 5
        for bl in range(8):
            loads[bl].wait()
            slot = bl % 3

            def retile(sx, carry, slot=slot, bl=bl):
                vals = [band[slot, c, y, pl.ds(sx * P, P)]
                        for c in range(C) for y in range(P)]
                for j, v in enumerate(vals):
                    sbuf[sx, bl, pl.ds(j * P, P)] = v
                return carry

            lax.fori_loop(0, G, retile, 0)
            if bl < 5:
                loads[bl + 3] = load(bl + 3, slot)

        pltpu.async_copy(
            sbuf, xp_hbm.at[pl.ds(sy * G, G), pl.ds(8 * bh, 8), :], sem_out
        ).wait()


_gather_sc = functools.partial(
    pl.kernel,
    mesh=plsc.VectorSubcoreMesh(core_axis_name="c", subcore_axis_name="s"),
    out_type=jax.ShapeDtypeStruct((S, B, PATCH), jnp.float32),
    scratch_types=[
        pltpu.VMEM((3, C, P, WID), jnp.float32),
        pltpu.VMEM((G, 8, PATCH), jnp.float32),
        pltpu.SemaphoreType.DMA,
        pltpu.SemaphoreType.DMA,
    ],
)(_gather_body)


def _matmul_body(x_ref, w_ref, b_ref, o_ref):
    acc = lax.dot_general(
        x_ref[...], w_ref[...], (((1,), (1,)), ((), ())),
        preferred_element_type=jnp.float32)
    o_ref[...] = acc + b_ref[...]


def _project_tc(x, wc, bias):
    m_blk = 1568
    grid = (ROWS // m_blk,)
    return pl.pallas_call(
        _matmul_body,
        grid=grid,
        in_specs=[
            pl.BlockSpec((m_blk, PATCH), lambda i: (i, 0)),
            pl.BlockSpec((PATCH, PATCH), lambda i: (0, 0)),
            pl.BlockSpec((1, PATCH), lambda i: (0, 0)),
        ],
        out_specs=pl.BlockSpec((m_blk, PATCH), lambda i: (i, 0)),
        out_shape=jax.ShapeDtypeStruct((ROWS, PATCH), jnp.float32),
    )(x, wc, bias)


def kernel(img, W, b):
    wc = W.reshape(-1, C, MAX_PIX)[:, :, : P * P].reshape(-1, PATCH)
    x = _gather_sc(img).reshape(ROWS, PATCH)
    out = _project_tc(x, wc, b.reshape(1, -1))
    return out.reshape(S, B, -1).transpose(1, 0, 2)
